# baseline (speedup 1.0000x reference)
import functools

import jax
import jax.numpy as jnp
from jax import lax
from jax.experimental import pallas as pl
from jax.experimental.pallas import tpu as pltpu
from jax.experimental.pallas import tpu_sc as plsc

MAX_POS = 100000
HALF = 32
EMB = 64
SEQ = 199
S1 = SEQ + 1
L = 16
NVEC = 13
PXW = NVEC * L
C0 = 104
C1 = SEQ - C0


def _make_pe_add(B, n_workers):
    rows_per_w = B // n_workers
    half_iters = rows_per_w // 2
    mesh = plsc.VectorSubcoreMesh(core_axis_name="c", subcore_axis_name="s")

    @functools.partial(
        pl.kernel,
        mesh=mesh,
        compiler_params=pltpu.CompilerParams(use_tc_tiling_on_sc=True),
        out_type=jax.ShapeDtypeStruct((B, S1, EMB), jnp.float32),
        scratch_types=[
            pltpu.VMEM((rows_per_w * PXW,), jnp.float32),
            pltpu.VMEM((NVEC * L,), jnp.int32),
            pltpu.VMEM((NVEC * L,), jnp.int32),
            pltpu.VMEM((NVEC * L + L,), jnp.int32),
            pltpu.VMEM((NVEC * L + L,), jnp.int32),
            pltpu.VMEM((S1, EMB), jnp.float32),
            pltpu.VMEM((S1, EMB), jnp.float32),
            pltpu.VMEM((S1, 128), jnp.float32),
            pltpu.VMEM((S1, 128), jnp.float32),
            pltpu.SemaphoreType.DMA,
            pltpu.SemaphoreType.DMA,
            pltpu.SemaphoreType.DMA,
            pltpu.SemaphoreType.DMA,
            pltpu.SemaphoreType.DMA,
            pltpu.SemaphoreType.DMA,
        ],
    )
    def pe_add(x_hbm, px_hbm, w_hbm, out_hbm,
               pxall_v, idx_a, idx_b, sub_a, sub_b,
               xrow_a, xrow_b, wbuf_a, wbuf_b,
               sem_xa, sem_xb, sem_ga, sem_gb, sem_oa, sem_ob):
        cid = lax.axis_index("c")
        sid = lax.axis_index("s")
        wid = sid * 2 + cid
        base = wid * rows_per_w

        pltpu.sync_copy(px_hbm.at[pl.ds(base * PXW, rows_per_w * PXW)],
                        pxall_v)
        pltpu.sync_copy(w_hbm.at[0], wbuf_a.at[0])
        pltpu.sync_copy(w_hbm.at[0], wbuf_b.at[0])

        def issue_row(r, il, idx_v, sub_v, xrow_v, wbuf_v, sem_x, sem_g):
            cp_x = pltpu.make_async_copy(x_hbm.at[r], xrow_v, sem_x)
            cp_x.start()
            for k in range(NVEC):
                v = pxall_v[pl.ds(il * PXW + k * L, L)]
                q = v / 20.0
                t = q.astype(jnp.int32)
                c = jnp.where(v > t.astype(jnp.float32) * 20.0, t + 1, t)
                idx_v[pl.ds(k * L, L)] = c >> 1
                sub_v[pl.ds(k * L, L)] = (c & 1) * EMB
            cp_g0 = pltpu.make_async_copy(
                w_hbm.at[idx_v.at[pl.ds(0, C0)]],
                wbuf_v.at[pl.ds(1, C0)], sem_g)
            cp_g1 = pltpu.make_async_copy(
                w_hbm.at[idx_v.at[pl.ds(C0, C1)]],
                wbuf_v.at[pl.ds(C0 + 1, C1)], sem_g)
            cp_g0.start()
            cp_g1.start()
            return (cp_x, cp_g0, cp_g1)

        def finish_row(r, handles, sub_v, xrow_v, wbuf_v, sem_o):
            cp_x, cp_g0, cp_g1 = handles
            cp_g0.wait()
            cp_g1.wait()
            cp_x.wait()
            for k in range(4):
                plsc.addupdate(xrow_v.at[0, pl.ds(k * L, L)],
                               wbuf_v[0, pl.ds(k * L, L)])

            def add_body(t, c2):
                tt = t + 1
                sub = sub_v[pl.ds(t, L)][0]
                for k in range(4):
                    plsc.addupdate(xrow_v.at[tt, pl.ds(k * L, L)],
                                   wbuf_v[tt, pl.ds(sub + k * L, L)])
                return c2

            lax.fori_loop(0, SEQ, add_body, 0)
            cp_o = pltpu.make_async_copy(xrow_v, out_hbm.at[r], sem_o)
            cp_o.start()

        def pair_body(j, carry):
            r0 = base + 2 * j
            r1 = r0 + 1

            @pl.when(j > 0)
            def _():
                pltpu.make_async_copy(xrow_a, out_hbm.at[r0], sem_oa).wait()
                pltpu.make_async_copy(xrow_b, out_hbm.at[r1], sem_ob).wait()

            ha = issue_row(r0, 2 * j, idx_a, sub_a, xrow_a, wbuf_a,
                           sem_xa, sem_ga)
            hb = issue_row(r1, 2 * j + 1, idx_b, sub_b, xrow_b, wbuf_b,
                           sem_xb, sem_gb)
            finish_row(r0, ha, sub_a, xrow_a, wbuf_a, sem_oa)
            finish_row(r1, hb, sub_b, xrow_b, wbuf_b, sem_ob)
            return carry

        lax.fori_loop(0, half_iters, pair_body, 0)
        rl0 = base + rows_per_w - 2
        pltpu.make_async_copy(xrow_a, out_hbm.at[rl0], sem_oa).wait()
        pltpu.make_async_copy(xrow_b, out_hbm.at[rl0 + 1], sem_ob).wait()

    return pe_add


def kernel(x, px, py, pex_w, pey_w):
    del py
    B = x.shape[0]
    info = plsc.get_sparse_core_info()
    n_workers = info.num_cores * info.num_subcores
    w = jnp.concatenate([pex_w, pey_w], axis=1).reshape(MAX_POS // 2, 128)
    px_p = jnp.pad(px, ((0, 0), (0, PXW - SEQ))).reshape(B * PXW)
    pe_add = _make_pe_add(B, n_workers)
    return pe_add(x, px_p, w)


# speedup vs baseline: 3.2906x; 3.2906x over previous
import functools

import jax
import jax.numpy as jnp
from jax import lax
from jax.experimental import pallas as pl
from jax.experimental.pallas import tpu as pltpu
from jax.experimental.pallas import tpu_sc as plsc

MAX_POS = 100000
HALF = 32
EMB = 64
SEQ = 199
S1 = SEQ + 1
ROW = S1 * EMB
L = 16
NVEC = 13
PXW = NVEC * L
BLK = 128
RS = 208
RW = RS * EMB


def _make_gather(B, n_workers):
    rows_per_w = B // n_workers
    half_iters = rows_per_w // 2
    mesh = plsc.VectorSubcoreMesh(core_axis_name="c", subcore_axis_name="s")

    @functools.partial(
        pl.kernel,
        mesh=mesh,
        compiler_params=pltpu.CompilerParams(use_tc_tiling_on_sc=False),
        out_type=jax.ShapeDtypeStruct((B * RS, EMB), jnp.float32),
        scratch_types=[
            pltpu.VMEM((rows_per_w * PXW,), jnp.float32),
            pltpu.VMEM((NVEC * L,), jnp.int32),
            pltpu.VMEM((NVEC * L,), jnp.int32),
            pltpu.VMEM((S1, EMB), jnp.float32),
            pltpu.VMEM((S1, EMB), jnp.float32),
            pltpu.SemaphoreType.DMA,
            pltpu.SemaphoreType.DMA,
            pltpu.SemaphoreType.DMA,
            pltpu.SemaphoreType.DMA,
        ],
    )
    def gather_pe(px_hbm, w_hbm, pe_hbm,
                  pxall_v, idx_a, idx_b, pe_a, pe_b,
                  sem_ga, sem_gb, sem_oa, sem_ob):
        cid = lax.axis_index("c")
        sid = lax.axis_index("s")
        wid = sid * 2 + cid
        base = wid * rows_per_w

        pltpu.sync_copy(px_hbm.at[pl.ds(base * PXW, rows_per_w * PXW)],
                        pxall_v)
        pltpu.sync_copy(w_hbm.at[0], pe_a.at[0])
        pltpu.sync_copy(w_hbm.at[0], pe_b.at[0])

        def issue_row(il, idx_v, pe_v, sem_g):
            for k in range(NVEC):
                v = pxall_v[pl.ds(il * PXW + k * L, L)]
                q = v / 20.0
                t = q.astype(jnp.int32)
                c = jnp.where(v > t.astype(jnp.float32) * 20.0, t + 1, t)
                idx_v[pl.ds(k * L, L)] = c
            cp_g0 = pltpu.make_async_copy(
                w_hbm.at[idx_v.at[pl.ds(0, 128)]],
                pe_v.at[pl.ds(1, 128)], sem_g)
            cp_g1 = pltpu.make_async_copy(
                w_hbm.at[idx_v.at[pl.ds(128, SEQ - 128)]],
                pe_v.at[pl.ds(129, SEQ - 128)], sem_g)
            cp_g0.start()
            cp_g1.start()
            return (cp_g0, cp_g1)

        def finish_row(r, handles, pe_v, sem_o):
            cp_g0, cp_g1 = handles
            cp_g0.wait()
            cp_g1.wait()
            pltpu.make_async_copy(
                pe_v, pe_hbm.at[pl.ds(r * RS, S1)], sem_o).start()

        def pair_body(j, carry):
            r0 = base + 2 * j
            r1 = r0 + 1

            @pl.when(j > 0)
            def _():
                pltpu.make_async_copy(
                    pe_a, pe_hbm.at[pl.ds(r0 * RS, S1)], sem_oa).wait()
                pltpu.make_async_copy(
                    pe_b, pe_hbm.at[pl.ds(r1 * RS, S1)], sem_ob).wait()

            ha = issue_row(2 * j, idx_a, pe_a, sem_ga)
            hb = issue_row(2 * j + 1, idx_b, pe_b, sem_gb)
            finish_row(r0, ha, pe_a, sem_oa)
            finish_row(r1, hb, pe_b, sem_ob)
            return carry

        lax.fori_loop(0, half_iters, pair_body, 0)
        rl0 = base + rows_per_w - 2
        pltpu.make_async_copy(
            pe_a, pe_hbm.at[pl.ds(rl0 * RS, S1)], sem_oa).wait()
        pltpu.make_async_copy(
            pe_b, pe_hbm.at[pl.ds((rl0 + 1) * RS, S1)], sem_ob).wait()

    return gather_pe


def _tc_add_body(xt_ref, pe_ref, out_ref):
    pvv = pe_ref[...].reshape(BLK * RW // 128, 128).reshape(
        BLK, RW // 128, 128)
    eye = (lax.broadcasted_iota(jnp.int32, (BLK, BLK), 0)
           == lax.broadcasted_iota(jnp.int32, (BLK, BLK), 1)
           ).astype(jnp.float32)
    for q in range(ROW // 128):
        pq = pvv[:, q, :]
        pqt = lax.dot_general(pq, eye, (((0,), (0,)), ((), ())),
                              precision=lax.Precision.HIGHEST)
        out_ref[pl.ds(2 * q, 2), :, :] = (
            xt_ref[pl.ds(2 * q, 2), :, :] + pqt.reshape(2, EMB, BLK))


def _tc_add(xt, pe, B):
    grid = B // BLK
    return pl.pallas_call(
        _tc_add_body,
        grid=(grid,),
        in_specs=[
            pl.BlockSpec((S1, EMB, BLK), lambda i: (0, 0, i)),
            pl.BlockSpec((BLK * RW,), lambda i: (i,)),
        ],
        out_specs=pl.BlockSpec((S1, EMB, BLK), lambda i: (0, 0, i)),
        out_shape=jax.ShapeDtypeStruct((S1, EMB, B), jnp.float32),
    )(xt, pe)


def kernel(x, px, py, pex_w, pey_w):
    del py
    B = x.shape[0]
    info = plsc.get_sparse_core_info()
    n_workers = info.num_cores * info.num_subcores
    w64 = jnp.concatenate([pex_w, pey_w], axis=1)
    px_p = jnp.pad(px, ((0, 0), (0, PXW - SEQ))).reshape(B * PXW)
    gather_pe = _make_gather(B, n_workers)
    pe = gather_pe(px_p, w64).reshape(B * RW)
    xt = jnp.transpose(x, (1, 2, 0))
    out_t = _tc_add(xt, pe, B)
    return jnp.transpose(out_t, (2, 0, 1))


# speedup vs baseline: 3.4337x; 1.0435x over previous
import functools

import jax
import jax.numpy as jnp
from jax import lax
from jax.experimental import pallas as pl
from jax.experimental.pallas import tpu as pltpu
from jax.experimental.pallas import tpu_sc as plsc

MAX_POS = 100000
HALF = 32
EMB = 64
SEQ = 199
S1 = SEQ + 1
L = 16
NVEC = 13
PXW = NVEC * L
BLK = 128
RS = 224
RW = RS * HALF
QT = S1 // 4


def _make_gather(B, n_workers):
    rows_per_w = B // n_workers
    half_iters = rows_per_w // 2
    mesh = plsc.VectorSubcoreMesh(core_axis_name="c", subcore_axis_name="s")

    @functools.partial(
        pl.kernel,
        mesh=mesh,
        compiler_params=pltpu.CompilerParams(use_tc_tiling_on_sc=False),
        out_type=(jax.ShapeDtypeStruct((B * RS, HALF), jnp.float32),
                  jax.ShapeDtypeStruct((B * RS, HALF), jnp.float32)),
        scratch_types=[
            pltpu.VMEM((rows_per_w * PXW,), jnp.float32),
            pltpu.VMEM((NVEC * L,), jnp.int32),
            pltpu.VMEM((NVEC * L,), jnp.int32),
            pltpu.VMEM((S1, HALF), jnp.float32),
            pltpu.VMEM((S1, HALF), jnp.float32),
            pltpu.VMEM((S1, HALF), jnp.float32),
            pltpu.VMEM((S1, HALF), jnp.float32),
            pltpu.SemaphoreType.DMA,
            pltpu.SemaphoreType.DMA,
            pltpu.SemaphoreType.DMA,
            pltpu.SemaphoreType.DMA,
        ],
    )
    def gather_pe(px_hbm, wx_hbm, wy_hbm, pex_hbm, pey_hbm,
                  pxall_v, idx_a, idx_b, pxr_a, pxr_b, pyr_a, pyr_b,
                  sem_ga, sem_gb, sem_oa, sem_ob):
        cid = lax.axis_index("c")
        sid = lax.axis_index("s")
        wid = sid * 2 + cid
        base = wid * rows_per_w

        pltpu.sync_copy(px_hbm.at[pl.ds(base * PXW, rows_per_w * PXW)],
                        pxall_v)
        pltpu.sync_copy(wx_hbm.at[0], pxr_a.at[0])
        pltpu.sync_copy(wx_hbm.at[0], pxr_b.at[0])
        pltpu.sync_copy(wy_hbm.at[0], pyr_a.at[0])
        pltpu.sync_copy(wy_hbm.at[0], pyr_b.at[0])

        def issue_row(il, idx_v, pxr_v, pyr_v, sem_g):
            for k in range(NVEC):
                v = pxall_v[pl.ds(il * PXW + k * L, L)]
                q = v / 20.0
                t = q.astype(jnp.int32)
                c = jnp.where(v > t.astype(jnp.float32) * 20.0, t + 1, t)
                idx_v[pl.ds(k * L, L)] = c
            cps = [
                pltpu.make_async_copy(
                    wx_hbm.at[idx_v.at[pl.ds(0, 128)]],
                    pxr_v.at[pl.ds(1, 128)], sem_g),
                pltpu.make_async_copy(
                    wx_hbm.at[idx_v.at[pl.ds(128, SEQ - 128)]],
                    pxr_v.at[pl.ds(129, SEQ - 128)], sem_g),
                pltpu.make_async_copy(
                    wy_hbm.at[idx_v.at[pl.ds(0, 128)]],
                    pyr_v.at[pl.ds(1, 128)], sem_g),
                pltpu.make_async_copy(
                    wy_hbm.at[idx_v.at[pl.ds(128, SEQ - 128)]],
                    pyr_v.at[pl.ds(129, SEQ - 128)], sem_g),
            ]
            for cp in cps:
                cp.start()
            return cps

        def finish_row(r, cps, pxr_v, pyr_v, sem_o):
            for cp in cps:
                cp.wait()
            pltpu.make_async_copy(
                pxr_v, pex_hbm.at[pl.ds(r * RS, S1)], sem_o).start()
            pltpu.make_async_copy(
                pyr_v, pey_hbm.at[pl.ds(r * RS, S1)], sem_o).start()

        def drain(r, pxr_v, pyr_v, sem_o):
            pltpu.make_async_copy(
                pxr_v, pex_hbm.at[pl.ds(r * RS, S1)], sem_o).wait()
            pltpu.make_async_copy(
                pyr_v, pey_hbm.at[pl.ds(r * RS, S1)], sem_o).wait()

        def pair_body(j, carry):
            r0 = base + 2 * j
            r1 = r0 + 1

            @pl.when(j > 0)
            def _():
                drain(r0, pxr_a, pyr_a, sem_oa)
                drain(r1, pxr_b, pyr_b, sem_ob)

            ha = issue_row(2 * j, idx_a, pxr_a, pyr_a, sem_ga)
            hb = issue_row(2 * j + 1, idx_b, pxr_b, pyr_b, sem_gb)
            finish_row(r0, ha, pxr_a, pyr_a, sem_oa)
            finish_row(r1, hb, pxr_b, pyr_b, sem_ob)
            return carry

        lax.fori_loop(0, half_iters, pair_body, 0)
        rl0 = base + rows_per_w - 2
        drain(rl0, pxr_a, pyr_a, sem_oa)
        drain(rl0 + 1, pxr_b, pyr_b, sem_ob)

    return gather_pe


def _tc_add_body(xt_ref, pex_ref, pey_ref, out_ref):
    pvx = pex_ref[...].reshape(BLK * RW // 128, 128).reshape(
        BLK, RW // 128, 128)
    pvy = pey_ref[...].reshape(BLK * RW // 128, 128).reshape(
        BLK, RW // 128, 128)
    eye = (lax.broadcasted_iota(jnp.int32, (BLK, BLK), 0)
           == lax.broadcasted_iota(jnp.int32, (BLK, BLK), 1)
           ).astype(jnp.float32)
    for q in range(QT):
        pqx = pvx[:, q, :]
        pqy = pvy[:, q, :]
        pqxt = lax.dot_general(pqx, eye, (((0,), (0,)), ((), ())),
                               precision=lax.Precision.HIGHEST)
        pqyt = lax.dot_general(pqy, eye, (((0,), (0,)), ((), ())),
                               precision=lax.Precision.HIGHEST)
        s0 = 4 * q
        out_ref[pl.ds(s0, 4), pl.ds(0, HALF), :] = (
            xt_ref[pl.ds(s0, 4), pl.ds(0, HALF), :]
            + pqxt.reshape(4, HALF, BLK))
        out_ref[pl.ds(s0, 4), pl.ds(HALF, HALF), :] = (
            xt_ref[pl.ds(s0, 4), pl.ds(HALF, HALF), :]
            + pqyt.reshape(4, HALF, BLK))


def _tc_add(xt, pex, pey, B):
    grid = B // BLK
    return pl.pallas_call(
        _tc_add_body,
        grid=(grid,),
        in_specs=[
            pl.BlockSpec((S1, EMB, BLK), lambda i: (0, 0, i)),
            pl.BlockSpec((BLK * RW,), lambda i: (i,)),
            pl.BlockSpec((BLK * RW,), lambda i: (i,)),
        ],
        out_specs=pl.BlockSpec((S1, EMB, BLK), lambda i: (0, 0, i)),
        out_shape=jax.ShapeDtypeStruct((S1, EMB, B), jnp.float32),
    )(xt, pex, pey)


def kernel(x, px, py, pex_w, pey_w):
    del py
    B = x.shape[0]
    info = plsc.get_sparse_core_info()
    n_workers = info.num_cores * info.num_subcores
    px_p = jnp.pad(px, ((0, 0), (0, PXW - SEQ))).reshape(B * PXW)
    gather_pe = _make_gather(B, n_workers)
    pex, pey = gather_pe(px_p, pex_w, pey_w)
    pex = pex.reshape(B * RW)
    pey = pey.reshape(B * RW)
    xt = jnp.transpose(x, (1, 2, 0))
    out_t = _tc_add(xt, pex, pey, B)
    return jnp.transpose(out_t, (2, 0, 1))
